# Initial kernel scaffold; baseline (speedup 1.0000x reference)
import jax
import jax.numpy as jnp
from jax.experimental import pallas as pl


def kernel(x, edge_index, W_l, b_l, W_r):
    raise NotImplementedError("write your pallas kernel here")


# speedup vs baseline: 4.5396x; 4.5396x over previous
import functools

import jax
import jax.numpy as jnp
from jax import lax
from jax.experimental import pallas as pl
from jax.experimental.pallas import tpu as pltpu
from jax.experimental.pallas import tpu_sc as plsc

N = 10000
D = 128
E = 320000

NC = 2
NS = 16
NW = NC * NS
L = 16

K = 128
CHUNKS = 80
PER_TILE = CHUNKS * K
E_PAD = NW * PER_TILE
C_ROWS = E_PAD // K

ACC_ROWS = 10240
PAD_DST = 10008
HN = 10240


def _sc_body(x_hbm, srcm, dstm, sums_out, counts_out,
             srcv, dstv, rows, hist, acc, sem):
    cid = lax.axis_index("c")
    sid = lax.axis_index("s")
    wid = sid * NC + cid

    zero16 = jnp.zeros((L,), jnp.float32)
    ones16 = jnp.ones((L,), jnp.float32)

    def z_rows(i, c):
        rows[i // 8, pl.ds((i % 8) * L, L)] = zero16
        return c
    lax.fori_loop(0, 128 * 8, z_rows, 0)

    def z_hist(i, c):
        hist[pl.ds(i * L, L)] = zero16
        return c
    lax.fori_loop(0, HN // L, z_hist, 0)

    for b in range(5):
        pltpu.sync_copy(rows, acc.at[pl.ds(sid * 640 + b * 128, 128)])

    pltpu.sync_copy(srcm.at[pl.ds(wid * CHUNKS, CHUNKS)], srcv)
    pltpu.sync_copy(dstm.at[pl.ds(wid * CHUNKS, CHUNKS)], dstv)

    plsc.subcore_barrier()

    def chunk(c, carry):
        cp = pltpu.async_copy(x_hbm.at[srcv.at[c]], rows, sem)
        for j in range(K // L):
            d = dstv[c, pl.ds(j * L, L)]
            plsc.addupdate_scatter(hist, [d], ones16)
        cp.wait()
        pltpu.sync_copy(rows, acc.at[dstv.at[c]], add=True)
        return carry
    lax.fori_loop(0, CHUNKS, chunk, 0)

    pltpu.sync_copy(hist, counts_out.at[cid, sid])

    plsc.subcore_barrier()

    pltpu.sync_copy(acc.at[pl.ds(sid * 624, 624)],
                    sums_out.at[cid, pl.ds(sid * 624, 624)])

    @pl.when(sid == 0)
    def _():
        pltpu.sync_copy(acc.at[pl.ds(16 * 624, 16)],
                        sums_out.at[cid, pl.ds(16 * 624, 16)])


@functools.partial(jax.jit, donate_argnums=())
def _sc_call(x, srcm, dstm):
    mesh = plsc.VectorSubcoreMesh(core_axis_name="c", subcore_axis_name="s")
    f = pl.kernel(
        _sc_body,
        out_type=(
            jax.ShapeDtypeStruct((NC, N, D), jnp.float32),
            jax.ShapeDtypeStruct((NC, NS, HN), jnp.float32),
        ),
        mesh=mesh,
        scratch_types=[
            pltpu.VMEM((CHUNKS, K), jnp.int32),
            pltpu.VMEM((CHUNKS, K), jnp.int32),
            pltpu.VMEM((K, D), jnp.float32),
            pltpu.VMEM((HN,), jnp.float32),
            pltpu.VMEM_SHARED((ACC_ROWS, D), jnp.float32),
            pltpu.SemaphoreType.DMA,
        ],
        compiler_params=pltpu.CompilerParams(needs_layout_passes=False),
    )
    return f(x, srcm, dstm)


def _cnt_body(h_ref, out_ref):
    s = jnp.sum(h_ref[...], axis=0, keepdims=True)
    out_ref[...] = 1.0 / jnp.maximum(s, 1.0)


def _cnt_call(h):
    return pl.pallas_call(
        _cnt_body,
        out_shape=jax.ShapeDtypeStruct((1, HN), jnp.float32),
    )(h)


def _tc_body(x_ref, s0, s1, r0, wl, wr, bl, out_ref):
    aggr = (s0[...] + s1[...]) * r0[...]
    y = lax.dot_general(aggr, wl[...], (((1,), (1,)), ((), ())),
                        preferred_element_type=jnp.float32)
    y = y + lax.dot_general(x_ref[...], wr[...], (((1,), (1,)), ((), ())),
                            preferred_element_type=jnp.float32)
    y = y + bl[...]
    g = 0.5 * y * (1.0 + lax.erf(y * 0.7071067811865476))
    out_ref[...] = x_ref[...] + g


def _tc_call(x, s0, s1, r0, W_l, W_r, bl):
    B = 2000
    grid = (N // B,)
    row_spec = pl.BlockSpec((B, D), lambda i: (i, 0))
    cnt_spec = pl.BlockSpec((B, 1), lambda i: (i, 0))
    w_spec = pl.BlockSpec((D, D), lambda i: (0, 0))
    b_spec = pl.BlockSpec((1, D), lambda i: (0, 0))
    return pl.pallas_call(
        _tc_body,
        grid=grid,
        in_specs=[row_spec, row_spec, row_spec, cnt_spec,
                  w_spec, w_spec, b_spec],
        out_specs=row_spec,
        out_shape=jax.ShapeDtypeStruct((N, D), jnp.float32),
    )(x, s0, s1, r0, W_l, W_r, bl)


def kernel(x, edge_index, W_l, b_l, W_r):
    src = edge_index[0]
    dst = edge_index[1]
    pad = E_PAD - E
    src_p = jnp.concatenate([src, jnp.zeros((pad,), jnp.int32)]).reshape(C_ROWS, K)
    dst_p = jnp.concatenate([dst, jnp.full((pad,), PAD_DST, jnp.int32)]).reshape(C_ROWS, K)
    sums_p, counts_p = _sc_call(x, src_p, dst_p)
    recip = _cnt_call(counts_p.reshape(NC * NS, HN))
    r0 = recip.reshape(-1)[:N].reshape(N, 1)
    return _tc_call(x, sums_p[0], sums_p[1], r0, W_l, W_r,
                    b_l.reshape(1, D))
